# initial kernel scaffold (unmeasured)
import functools

import numpy as np

import jax
import jax.numpy as jnp
from jax import lax
from jax.experimental import pallas as pl
from jax.experimental.pallas import tpu as pltpu

N_DEV = 4
SQ = 2048
D = 1024
DH = 128
H_LOC = 8
SCALE = 0.08838834764831843

_inv = 1.0 / (10000.0 ** (np.arange(0, DH, 2) / DH))
_pos = np.arange(SQ)[:, None] * _inv[None, :]
_COS = np.repeat(np.cos(_pos), 2, axis=-1).astype(np.float32)
_SIN = np.repeat(np.sin(_pos), 2, axis=-1).astype(np.float32)
_ROT = np.zeros((DH, DH), dtype=np.float32)
for _k in range(DH // 2):
    _ROT[2 * _k + 1, 2 * _k] = -1.0
    _ROT[2 * _k, 2 * _k + 1] = 1.0


def _ag_body(x_ref, out_ref, comm_ref, send_sems, recv_sems):
    my = lax.axis_index("i")
    left = lax.rem(my + 3, N_DEV)
    right = lax.rem(my + 1, N_DEV)

    barrier = pltpu.get_barrier_semaphore()
    for nbr in (left, right):
        pl.semaphore_signal(
            barrier, inc=1, device_id=(nbr,),
            device_id_type=pl.DeviceIdType.MESH,
        )
    pl.semaphore_wait(barrier, 2)

    out_ref[pl.ds(my * SQ, SQ), :] = x_ref[:]
    comm_ref[0, :, :] = x_ref[:]

    for h in range(N_DEV - 1):
        rdma = pltpu.make_async_remote_copy(
            src_ref=comm_ref.at[h],
            dst_ref=comm_ref.at[h + 1],
            send_sem=send_sems.at[h],
            recv_sem=recv_sems.at[h],
            device_id=(right,),
            device_id_type=pl.DeviceIdType.MESH,
        )
        rdma.start()
        rdma.wait()
        origin = lax.rem(my + 3 - h, N_DEV)
        out_ref[pl.ds(origin * SQ, SQ), :] = comm_ref[h + 1, :, :]


def _all_gather(x2):
    return pl.pallas_call(
        _ag_body,
        out_shape=jax.ShapeDtypeStruct((N_DEV * SQ, D), jnp.bfloat16),
        in_specs=[pl.BlockSpec(memory_space=pltpu.VMEM)],
        out_specs=pl.BlockSpec(memory_space=pltpu.VMEM),
        scratch_shapes=[
            pltpu.VMEM((N_DEV, SQ, D), jnp.bfloat16),
            pltpu.SemaphoreType.DMA((N_DEV - 1,)),
            pltpu.SemaphoreType.DMA((N_DEV - 1,)),
        ],
        compiler_params=pltpu.CompilerParams(collective_id=0),
    )(x2)


def _attn_body(x_ref, wq_ref, wk_ref, wv_ref, wo_ref, cos_ref, sin_ref,
               rot_ref, out_ref):
    xb = x_ref[:]
    cos = cos_ref[:]
    sin = sin_ref[:]
    rot = rot_ref[:]

    qm = jnp.dot(xb, wq_ref[:], preferred_element_type=jnp.float32)
    km = jnp.dot(xb, wk_ref[:], preferred_element_type=jnp.float32)
    vm = jnp.dot(xb, wv_ref[:], preferred_element_type=jnp.float32)
    vm = vm.astype(jnp.bfloat16)

    ctx = []
    for h in range(H_LOC):
        sl = slice(h * DH, (h + 1) * DH)
        q = qm[:, sl]
        k = km[:, sl]
        qrot = jnp.dot(q.astype(jnp.bfloat16), rot,
                       preferred_element_type=jnp.float32)
        krot = jnp.dot(k.astype(jnp.bfloat16), rot,
                       preferred_element_type=jnp.float32)
        qr = (q * cos + qrot * sin).astype(jnp.bfloat16)
        kr = (k * cos + krot * sin).astype(jnp.bfloat16)
        s = lax.dot_general(qr, kr, (((1,), (1,)), ((), ())),
                            preferred_element_type=jnp.float32)
        s = s * SCALE
        s = s - jnp.max(s, axis=-1, keepdims=True)
        w = jnp.exp(s)
        w = w / jnp.sum(w, axis=-1, keepdims=True)
        ctx.append(jnp.dot(w.astype(jnp.bfloat16), vm[:, sl],
                           preferred_element_type=jnp.float32)
                   .astype(jnp.bfloat16))
    ctxm = jnp.concatenate(ctx, axis=1)
    out_ref[:] = jnp.dot(ctxm, wo_ref[:],
                         preferred_element_type=jnp.float32).astype(jnp.bfloat16)


def _attention(xg, wq, wk, wv, wo):
    cos = jnp.asarray(_COS)
    sin = jnp.asarray(_SIN)
    rot = jnp.asarray(_ROT, dtype=jnp.bfloat16)
    full = lambda shape: pl.BlockSpec(shape, lambda b: (0,) * len(shape))
    return pl.pallas_call(
        _attn_body,
        grid=(N_DEV,),
        in_specs=[
            pl.BlockSpec((SQ, D), lambda b: (b, 0)),
            full((D, D)), full((D, D)), full((D, D)), full((D, D)),
            full((SQ, DH)), full((SQ, DH)), full((DH, DH)),
        ],
        out_specs=pl.BlockSpec((SQ, D), lambda b: (b, 0)),
        out_shape=jax.ShapeDtypeStruct((N_DEV * SQ, D), jnp.bfloat16),
    )(xg, wq, wk, wv, wo, cos, sin, rot)


def _rs_body(p_ref, out_ref, sbuf_ref, comm_ref, send_sems, recv_sems):
    my = lax.axis_index("i")
    left = lax.rem(my + 3, N_DEV)
    right = lax.rem(my + 1, N_DEV)

    barrier = pltpu.get_barrier_semaphore()
    for nbr in (left, right):
        pl.semaphore_signal(
            barrier, inc=1, device_id=(nbr,),
            device_id_type=pl.DeviceIdType.MESH,
        )
    pl.semaphore_wait(barrier, 2)

    for s in range(N_DEV - 1):
        if s == 0:
            d_send = lax.rem(my + 3, N_DEV)
            src = p_ref.at[pl.ds(d_send * SQ, SQ), :]
        else:
            src = sbuf_ref.at[s - 1]
        rdma = pltpu.make_async_remote_copy(
            src_ref=src,
            dst_ref=comm_ref.at[s],
            send_sem=send_sems.at[s],
            recv_sem=recv_sems.at[s],
            device_id=(right,),
            device_id_type=pl.DeviceIdType.MESH,
        )
        rdma.start()
        rdma.wait()
        d_recv = lax.rem(my + 2 - s, N_DEV)
        mine = p_ref[pl.ds(d_recv * SQ, SQ), :].astype(jnp.float32)
        acc = comm_ref[s, :, :].astype(jnp.float32) + mine
        if s < N_DEV - 2:
            sbuf_ref[s, :, :] = acc.astype(jnp.bfloat16)
        else:
            out_ref[:] = acc


def _reduce_scatter(partial):
    return pl.pallas_call(
        _rs_body,
        out_shape=jax.ShapeDtypeStruct((SQ, D), jnp.float32),
        in_specs=[pl.BlockSpec(memory_space=pltpu.VMEM)],
        out_specs=pl.BlockSpec(memory_space=pltpu.VMEM),
        scratch_shapes=[
            pltpu.VMEM((N_DEV - 2, SQ, D), jnp.bfloat16),
            pltpu.VMEM((N_DEV - 1, SQ, D), jnp.bfloat16),
            pltpu.SemaphoreType.DMA((N_DEV - 1,)),
            pltpu.SemaphoreType.DMA((N_DEV - 1,)),
        ],
        compiler_params=pltpu.CompilerParams(collective_id=1),
    )(partial)


def kernel(x, Wq, Wk, Wv, Wo):
    x2 = x.reshape(SQ, D).astype(jnp.bfloat16)
    xg = _all_gather(x2)
    partial = _attention(
        xg,
        Wq.astype(jnp.bfloat16),
        Wk.astype(jnp.bfloat16),
        Wv.astype(jnp.bfloat16),
        Wo.astype(jnp.bfloat16),
    )
    out = _reduce_scatter(partial)
    return out.reshape(1, SQ, D)


# baseline (device time: 853611 ns/iter reference)
import functools

import numpy as np

import jax
import jax.numpy as jnp
from jax import lax
from jax.experimental import pallas as pl
from jax.experimental.pallas import tpu as pltpu

N_DEV = 4
SQ = 2048
D = 1024
DH = 128
H_LOC = 8
SCALE = 0.08838834764831843

_inv = 1.0 / (10000.0 ** (np.arange(0, DH, 2) / DH))
_pos = np.arange(SQ)[:, None] * _inv[None, :]
_COS = np.repeat(np.cos(_pos), 2, axis=-1).astype(np.float32)
_SIN = np.repeat(np.sin(_pos), 2, axis=-1).astype(np.float32)
_ROT = np.zeros((DH, DH), dtype=np.float32)
for _k in range(DH // 2):
    _ROT[2 * _k + 1, 2 * _k] = -1.0
    _ROT[2 * _k, 2 * _k + 1] = 1.0


def _ag_body(x_ref, out_ref, comm_ref, send_sems, recv_sems):
    my = lax.axis_index("i")
    left = lax.rem(my + 3, N_DEV)
    right = lax.rem(my + 1, N_DEV)

    barrier = pltpu.get_barrier_semaphore()
    for nbr in (left, right):
        pl.semaphore_signal(
            barrier, inc=1, device_id=(nbr,),
            device_id_type=pl.DeviceIdType.MESH,
        )
    pl.semaphore_wait(barrier, 2)

    out_ref[pl.ds(my * SQ, SQ), :] = x_ref[:]
    comm_ref[0, :, :] = x_ref[:]

    for h in range(N_DEV - 1):
        rdma = pltpu.make_async_remote_copy(
            src_ref=comm_ref.at[h],
            dst_ref=comm_ref.at[h + 1],
            send_sem=send_sems.at[h],
            recv_sem=recv_sems.at[h],
            device_id=(right,),
            device_id_type=pl.DeviceIdType.MESH,
        )
        rdma.start()
        rdma.wait()
        origin = lax.rem(my + 3 - h, N_DEV)
        out_ref[pl.ds(origin * SQ, SQ), :] = comm_ref[h + 1, :, :]


def _all_gather(x2):
    return pl.pallas_call(
        _ag_body,
        out_shape=jax.ShapeDtypeStruct((N_DEV * SQ, D), jnp.bfloat16),
        in_specs=[pl.BlockSpec(memory_space=pltpu.VMEM)],
        out_specs=pl.BlockSpec(memory_space=pltpu.VMEM),
        scratch_shapes=[
            pltpu.VMEM((N_DEV, SQ, D), jnp.bfloat16),
            pltpu.SemaphoreType.DMA((N_DEV - 1,)),
            pltpu.SemaphoreType.DMA((N_DEV - 1,)),
        ],
        compiler_params=pltpu.CompilerParams(
            collective_id=0, vmem_limit_bytes=60 * 1024 * 1024,
        ),
    )(x2)


def _attn_body(x_ref, wq_ref, wk_ref, wv_ref, cos_ref, sin_ref, rot_ref,
               out_ref):
    xb = x_ref[:]
    cos = cos_ref[:]
    sin = sin_ref[:]
    rot = rot_ref[:]

    q = jnp.dot(xb, wq_ref[:], preferred_element_type=jnp.float32)
    k = jnp.dot(xb, wk_ref[:], preferred_element_type=jnp.float32)
    v = jnp.dot(xb, wv_ref[:], preferred_element_type=jnp.float32)
    v = v.astype(jnp.bfloat16)

    qrot = jnp.dot(q.astype(jnp.bfloat16), rot,
                   preferred_element_type=jnp.float32)
    krot = jnp.dot(k.astype(jnp.bfloat16), rot,
                   preferred_element_type=jnp.float32)
    qr = (q * cos + qrot * sin).astype(jnp.bfloat16)
    kr = (k * cos + krot * sin).astype(jnp.bfloat16)

    s = lax.dot_general(qr, kr, (((1,), (1,)), ((), ())),
                        preferred_element_type=jnp.float32)
    s = s * SCALE
    s = s - jnp.max(s, axis=-1, keepdims=True)
    w = jnp.exp(s)
    w = w / jnp.sum(w, axis=-1, keepdims=True)
    out_ref[:] = jnp.dot(w.astype(jnp.bfloat16), v,
                         preferred_element_type=jnp.float32).astype(jnp.bfloat16)


def _attention(xg, wq, wk, wv):
    cos = jnp.asarray(_COS)
    sin = jnp.asarray(_SIN)
    rot = jnp.asarray(_ROT, dtype=jnp.bfloat16)
    return pl.pallas_call(
        _attn_body,
        grid=(N_DEV, H_LOC),
        in_specs=[
            pl.BlockSpec((SQ, D), lambda b, h: (b, 0)),
            pl.BlockSpec((D, DH), lambda b, h: (0, h)),
            pl.BlockSpec((D, DH), lambda b, h: (0, h)),
            pl.BlockSpec((D, DH), lambda b, h: (0, h)),
            pl.BlockSpec((SQ, DH), lambda b, h: (0, 0)),
            pl.BlockSpec((SQ, DH), lambda b, h: (0, 0)),
            pl.BlockSpec((DH, DH), lambda b, h: (0, 0)),
        ],
        out_specs=pl.BlockSpec((SQ, DH), lambda b, h: (b, h)),
        out_shape=jax.ShapeDtypeStruct((N_DEV * SQ, D), jnp.bfloat16),
        compiler_params=pltpu.CompilerParams(
            vmem_limit_bytes=60 * 1024 * 1024,
        ),
    )(xg, wq, wk, wv, cos, sin, rot)


def _proj_body(c_ref, wo_ref, out_ref):
    out_ref[:] = jnp.dot(c_ref[:], wo_ref[:],
                         preferred_element_type=jnp.float32).astype(jnp.bfloat16)


def _project(ctx, wo):
    return pl.pallas_call(
        _proj_body,
        grid=(N_DEV,),
        in_specs=[
            pl.BlockSpec((SQ, D), lambda b: (b, 0)),
            pl.BlockSpec((D, D), lambda b: (0, 0)),
        ],
        out_specs=pl.BlockSpec((SQ, D), lambda b: (b, 0)),
        out_shape=jax.ShapeDtypeStruct((N_DEV * SQ, D), jnp.bfloat16),
        compiler_params=pltpu.CompilerParams(
            vmem_limit_bytes=60 * 1024 * 1024,
        ),
    )(ctx, wo)


def _rs_body(p_ref, out_ref, sbuf_ref, comm_ref, send_sems, recv_sems):
    my = lax.axis_index("i")
    left = lax.rem(my + 3, N_DEV)
    right = lax.rem(my + 1, N_DEV)

    barrier = pltpu.get_barrier_semaphore()
    for nbr in (left, right):
        pl.semaphore_signal(
            barrier, inc=1, device_id=(nbr,),
            device_id_type=pl.DeviceIdType.MESH,
        )
    pl.semaphore_wait(barrier, 2)

    for s in range(N_DEV - 1):
        if s == 0:
            d_send = lax.rem(my + 3, N_DEV)
            src = p_ref.at[pl.ds(d_send * SQ, SQ), :]
        else:
            src = sbuf_ref.at[s - 1]
        rdma = pltpu.make_async_remote_copy(
            src_ref=src,
            dst_ref=comm_ref.at[s],
            send_sem=send_sems.at[s],
            recv_sem=recv_sems.at[s],
            device_id=(right,),
            device_id_type=pl.DeviceIdType.MESH,
        )
        rdma.start()
        rdma.wait()
        d_recv = lax.rem(my + 2 - s, N_DEV)
        mine = p_ref[pl.ds(d_recv * SQ, SQ), :].astype(jnp.float32)
        acc = comm_ref[s, :, :].astype(jnp.float32) + mine
        if s < N_DEV - 2:
            sbuf_ref[s, :, :] = acc.astype(jnp.bfloat16)
        else:
            out_ref[:] = acc


def _reduce_scatter(partial):
    return pl.pallas_call(
        _rs_body,
        out_shape=jax.ShapeDtypeStruct((SQ, D), jnp.float32),
        in_specs=[pl.BlockSpec(memory_space=pltpu.VMEM)],
        out_specs=pl.BlockSpec(memory_space=pltpu.VMEM),
        scratch_shapes=[
            pltpu.VMEM((N_DEV - 2, SQ, D), jnp.bfloat16),
            pltpu.VMEM((N_DEV - 1, SQ, D), jnp.bfloat16),
            pltpu.SemaphoreType.DMA((N_DEV - 1,)),
            pltpu.SemaphoreType.DMA((N_DEV - 1,)),
        ],
        compiler_params=pltpu.CompilerParams(
            collective_id=1, vmem_limit_bytes=60 * 1024 * 1024,
        ),
    )(partial)


def kernel(x, Wq, Wk, Wv, Wo):
    x2 = x.reshape(SQ, D).astype(jnp.bfloat16)
    xg = _all_gather(x2)
    ctx = _attention(
        xg,
        Wq.astype(jnp.bfloat16),
        Wk.astype(jnp.bfloat16),
        Wv.astype(jnp.bfloat16),
    )
    partial = _project(ctx, Wo.astype(jnp.bfloat16))
    out = _reduce_scatter(partial)
    return out.reshape(1, SQ, D)


# device time: 754392 ns/iter; 1.1315x vs baseline; 1.1315x over previous
import functools

import numpy as np

import jax
import jax.numpy as jnp
from jax import lax
from jax.experimental import pallas as pl
from jax.experimental.pallas import tpu as pltpu

N_DEV = 4
SQ = 2048
D = 1024
DH = 128
H_LOC = 8
SCALE = 0.08838834764831843

_inv = 1.0 / (10000.0 ** (np.arange(0, DH, 2) / DH))
_pos = np.arange(SQ)[:, None] * _inv[None, :]
_COS = np.repeat(np.cos(_pos), 2, axis=-1).astype(np.float32)
_SIN = np.repeat(np.sin(_pos), 2, axis=-1).astype(np.float32)
_ROT = np.zeros((DH, DH), dtype=np.float32)
for _k in range(DH // 2):
    _ROT[2 * _k + 1, 2 * _k] = -1.0
    _ROT[2 * _k, 2 * _k + 1] = 1.0


def _ag_body(x_ref, out_ref, comm_ref, send_sems, recv_sems):
    my = lax.axis_index("i")
    left = lax.rem(my + 3, N_DEV)
    right = lax.rem(my + 1, N_DEV)

    barrier = pltpu.get_barrier_semaphore()
    for nbr in (left, right):
        pl.semaphore_signal(
            barrier, inc=1, device_id=(nbr,),
            device_id_type=pl.DeviceIdType.MESH,
        )
    pl.semaphore_wait(barrier, 2)

    out_ref[pl.ds(my * SQ, SQ), :] = x_ref[:]
    comm_ref[0, :, :] = x_ref[:]

    for h in range(N_DEV - 1):
        rdma = pltpu.make_async_remote_copy(
            src_ref=comm_ref.at[h],
            dst_ref=comm_ref.at[h + 1],
            send_sem=send_sems.at[h],
            recv_sem=recv_sems.at[h],
            device_id=(right,),
            device_id_type=pl.DeviceIdType.MESH,
        )
        rdma.start()
        rdma.wait()
        origin = lax.rem(my + 3 - h, N_DEV)
        out_ref[pl.ds(origin * SQ, SQ), :] = comm_ref[h + 1, :, :]


def _all_gather(x2):
    return pl.pallas_call(
        _ag_body,
        out_shape=jax.ShapeDtypeStruct((N_DEV * SQ, D), jnp.bfloat16),
        in_specs=[pl.BlockSpec(memory_space=pltpu.VMEM)],
        out_specs=pl.BlockSpec(memory_space=pltpu.VMEM),
        scratch_shapes=[
            pltpu.VMEM((N_DEV, SQ, D), jnp.bfloat16),
            pltpu.SemaphoreType.DMA((N_DEV - 1,)),
            pltpu.SemaphoreType.DMA((N_DEV - 1,)),
        ],
        compiler_params=pltpu.CompilerParams(
            collective_id=0, vmem_limit_bytes=60 * 1024 * 1024,
        ),
    )(x2)


def _attn_body(x_ref, wq_ref, wk_ref, wv_ref, cos_ref, sin_ref, rot_ref,
               out_ref):
    xb = x_ref[:]
    cos = cos_ref[:]
    sin = sin_ref[:]
    rot = rot_ref[:]

    q = jnp.dot(xb, wq_ref[:], preferred_element_type=jnp.float32)
    k = jnp.dot(xb, wk_ref[:], preferred_element_type=jnp.float32)
    v = jnp.dot(xb, wv_ref[:], preferred_element_type=jnp.float32)
    v = v.astype(jnp.bfloat16)

    qrot = jnp.dot(q.astype(jnp.bfloat16), rot,
                   preferred_element_type=jnp.float32)
    krot = jnp.dot(k.astype(jnp.bfloat16), rot,
                   preferred_element_type=jnp.float32)
    qr = (q * cos + qrot * sin).astype(jnp.bfloat16)
    kr = (k * cos + krot * sin).astype(jnp.bfloat16)

    s = lax.dot_general(qr, kr, (((1,), (1,)), ((), ())),
                        preferred_element_type=jnp.float32)
    w = jnp.exp(s * SCALE)
    denom = jnp.sum(w, axis=-1, keepdims=True)
    ctx = jnp.dot(w.astype(jnp.bfloat16), v,
                  preferred_element_type=jnp.float32)
    out_ref[:] = (ctx * (1.0 / denom)).astype(jnp.bfloat16)


def _attention(xg, wq, wk, wv):
    cos = jnp.asarray(_COS)
    sin = jnp.asarray(_SIN)
    rot = jnp.asarray(_ROT, dtype=jnp.bfloat16)
    return pl.pallas_call(
        _attn_body,
        grid=(N_DEV, H_LOC),
        in_specs=[
            pl.BlockSpec((SQ, D), lambda b, h: (b, 0)),
            pl.BlockSpec((D, DH), lambda b, h: (0, h)),
            pl.BlockSpec((D, DH), lambda b, h: (0, h)),
            pl.BlockSpec((D, DH), lambda b, h: (0, h)),
            pl.BlockSpec((SQ, DH), lambda b, h: (0, 0)),
            pl.BlockSpec((SQ, DH), lambda b, h: (0, 0)),
            pl.BlockSpec((DH, DH), lambda b, h: (0, 0)),
        ],
        out_specs=pl.BlockSpec((SQ, DH), lambda b, h: (b, h)),
        out_shape=jax.ShapeDtypeStruct((N_DEV * SQ, D), jnp.bfloat16),
        compiler_params=pltpu.CompilerParams(
            vmem_limit_bytes=60 * 1024 * 1024,
        ),
    )(xg, wq, wk, wv, cos, sin, rot)


def _proj_body(c_ref, wo_ref, out_ref):
    out_ref[:] = jnp.dot(c_ref[:], wo_ref[:],
                         preferred_element_type=jnp.float32).astype(jnp.bfloat16)


def _project(ctx, wo):
    return pl.pallas_call(
        _proj_body,
        grid=(N_DEV,),
        in_specs=[
            pl.BlockSpec((SQ, D), lambda b: (b, 0)),
            pl.BlockSpec((D, D), lambda b: (0, 0)),
        ],
        out_specs=pl.BlockSpec((SQ, D), lambda b: (b, 0)),
        out_shape=jax.ShapeDtypeStruct((N_DEV * SQ, D), jnp.bfloat16),
        compiler_params=pltpu.CompilerParams(
            vmem_limit_bytes=60 * 1024 * 1024,
        ),
    )(ctx, wo)


def _rs_body(p_ref, out_ref, sbuf_ref, comm_ref, send_sems, recv_sems):
    my = lax.axis_index("i")
    left = lax.rem(my + 3, N_DEV)
    right = lax.rem(my + 1, N_DEV)

    barrier = pltpu.get_barrier_semaphore()
    for nbr in (left, right):
        pl.semaphore_signal(
            barrier, inc=1, device_id=(nbr,),
            device_id_type=pl.DeviceIdType.MESH,
        )
    pl.semaphore_wait(barrier, 2)

    for s in range(N_DEV - 1):
        if s == 0:
            d_send = lax.rem(my + 3, N_DEV)
            src = p_ref.at[pl.ds(d_send * SQ, SQ), :]
        else:
            src = sbuf_ref.at[s - 1]
        rdma = pltpu.make_async_remote_copy(
            src_ref=src,
            dst_ref=comm_ref.at[s],
            send_sem=send_sems.at[s],
            recv_sem=recv_sems.at[s],
            device_id=(right,),
            device_id_type=pl.DeviceIdType.MESH,
        )
        rdma.start()
        rdma.wait()
        d_recv = lax.rem(my + 2 - s, N_DEV)
        mine = p_ref[pl.ds(d_recv * SQ, SQ), :].astype(jnp.float32)
        acc = comm_ref[s, :, :].astype(jnp.float32) + mine
        if s < N_DEV - 2:
            sbuf_ref[s, :, :] = acc.astype(jnp.bfloat16)
        else:
            out_ref[:] = acc


def _reduce_scatter(partial):
    return pl.pallas_call(
        _rs_body,
        out_shape=jax.ShapeDtypeStruct((SQ, D), jnp.float32),
        in_specs=[pl.BlockSpec(memory_space=pltpu.VMEM)],
        out_specs=pl.BlockSpec(memory_space=pltpu.VMEM),
        scratch_shapes=[
            pltpu.VMEM((N_DEV - 2, SQ, D), jnp.bfloat16),
            pltpu.VMEM((N_DEV - 1, SQ, D), jnp.bfloat16),
            pltpu.SemaphoreType.DMA((N_DEV - 1,)),
            pltpu.SemaphoreType.DMA((N_DEV - 1,)),
        ],
        compiler_params=pltpu.CompilerParams(
            collective_id=1, vmem_limit_bytes=60 * 1024 * 1024,
        ),
    )(partial)


def kernel(x, Wq, Wk, Wv, Wo):
    x2 = x.reshape(SQ, D).astype(jnp.bfloat16)
    xg = _all_gather(x2)
    ctx = _attention(
        xg,
        Wq.astype(jnp.bfloat16),
        Wk.astype(jnp.bfloat16),
        Wv.astype(jnp.bfloat16),
    )
    partial = _project(ctx, Wo.astype(jnp.bfloat16))
    out = _reduce_scatter(partial)
    return out.reshape(1, SQ, D)


# device time: 602638 ns/iter; 1.4165x vs baseline; 1.2518x over previous
import functools

import numpy as np

import jax
import jax.numpy as jnp
from jax import lax
from jax.experimental import pallas as pl
from jax.experimental.pallas import tpu as pltpu

N_DEV = 4
SQ = 2048
D = 1024
DH = 128
H_LOC = 8
SCALE = 0.08838834764831843

_inv = 1.0 / (10000.0 ** (np.arange(0, DH, 2) / DH))
_pos = np.arange(SQ)[:, None] * _inv[None, :]
_COS = np.repeat(np.cos(_pos), 2, axis=-1).astype(np.float32)
_SIN = np.repeat(np.sin(_pos), 2, axis=-1).astype(np.float32)
_ROT = np.zeros((DH, DH), dtype=np.float32)
for _k in range(DH // 2):
    _ROT[2 * _k + 1, 2 * _k] = -1.0
    _ROT[2 * _k, 2 * _k + 1] = 1.0


def _attn_body(x_ref, wq_ref, wk_ref, wv_ref, cos_ref, sin_ref, rot_ref,
               out_ref, comm_ref, send_sems, recv_sems):
    b = pl.program_id(0)
    h = pl.program_id(1)
    my = lax.axis_index("i")
    left = lax.rem(my + 3, N_DEV)
    right = lax.rem(my + 1, N_DEV)

    @pl.when(jnp.logical_and(b == 0, h == 0))
    def _():
        barrier = pltpu.get_barrier_semaphore()
        for nbr in (left, right):
            pl.semaphore_signal(
                barrier, inc=1, device_id=(nbr,),
                device_id_type=pl.DeviceIdType.MESH,
            )
        pl.semaphore_wait(barrier, 2)
        hop0 = pltpu.make_async_remote_copy(
            src_ref=x_ref,
            dst_ref=comm_ref.at[0],
            send_sem=send_sems.at[0],
            recv_sem=recv_sems.at[0],
            device_id=(right,),
            device_id_type=pl.DeviceIdType.MESH,
        )
        hop0.start()

    for bb in range(1, N_DEV):
        @pl.when(jnp.logical_and(b == bb, h == 0))
        def _(bb=bb):
            prev = pltpu.make_async_remote_copy(
                src_ref=comm_ref.at[bb - 1],
                dst_ref=comm_ref.at[bb - 1],
                send_sem=send_sems.at[bb - 1],
                recv_sem=recv_sems.at[bb - 1],
                device_id=(right,),
                device_id_type=pl.DeviceIdType.MESH,
            )
            prev.wait_send()
            prev.wait_recv()
            if bb <= N_DEV - 2:
                nxt = pltpu.make_async_remote_copy(
                    src_ref=comm_ref.at[bb - 1],
                    dst_ref=comm_ref.at[bb],
                    send_sem=send_sems.at[bb],
                    recv_sem=recv_sems.at[bb],
                    device_id=(right,),
                    device_id_type=pl.DeviceIdType.MESH,
                )
                nxt.start()

    xb = jnp.where(b == 0, x_ref[:], comm_ref[jnp.maximum(b - 1, 0)])
    cos = cos_ref[:]
    sin = sin_ref[:]
    rot = rot_ref[:]

    q = jnp.dot(xb, wq_ref[:], preferred_element_type=jnp.float32)
    k = jnp.dot(xb, wk_ref[:], preferred_element_type=jnp.float32)
    v = jnp.dot(xb, wv_ref[:], preferred_element_type=jnp.float32)
    v = v.astype(jnp.bfloat16)

    qrot = jnp.dot(q.astype(jnp.bfloat16), rot,
                   preferred_element_type=jnp.float32)
    krot = jnp.dot(k.astype(jnp.bfloat16), rot,
                   preferred_element_type=jnp.float32)
    qr = (q * cos + qrot * sin).astype(jnp.bfloat16)
    kr = (k * cos + krot * sin).astype(jnp.bfloat16)

    s = lax.dot_general(qr, kr, (((1,), (1,)), ((), ())),
                        preferred_element_type=jnp.float32)
    w = jnp.exp(s * SCALE)
    denom = jnp.sum(w, axis=-1, keepdims=True)
    ctx = jnp.dot(w.astype(jnp.bfloat16), v,
                  preferred_element_type=jnp.float32)
    out_ref[:] = (ctx * (1.0 / denom)).astype(jnp.bfloat16)


def _attention(x2, wq, wk, wv):
    cos = jnp.asarray(_COS)
    sin = jnp.asarray(_SIN)
    rot = jnp.asarray(_ROT, dtype=jnp.bfloat16)
    return pl.pallas_call(
        _attn_body,
        grid=(N_DEV, H_LOC),
        in_specs=[
            pl.BlockSpec((SQ, D), lambda b, h: (0, 0)),
            pl.BlockSpec((D, DH), lambda b, h: (0, h)),
            pl.BlockSpec((D, DH), lambda b, h: (0, h)),
            pl.BlockSpec((D, DH), lambda b, h: (0, h)),
            pl.BlockSpec((SQ, DH), lambda b, h: (0, 0)),
            pl.BlockSpec((SQ, DH), lambda b, h: (0, 0)),
            pl.BlockSpec((DH, DH), lambda b, h: (0, 0)),
        ],
        out_specs=pl.BlockSpec((SQ, DH), lambda b, h: (b, h)),
        out_shape=jax.ShapeDtypeStruct((N_DEV * SQ, D), jnp.bfloat16),
        scratch_shapes=[
            pltpu.VMEM((N_DEV - 1, SQ, D), jnp.bfloat16),
            pltpu.SemaphoreType.DMA((N_DEV - 1,)),
            pltpu.SemaphoreType.DMA((N_DEV - 1,)),
        ],
        compiler_params=pltpu.CompilerParams(
            collective_id=0, vmem_limit_bytes=60 * 1024 * 1024,
        ),
    )(x2, wq, wk, wv, cos, sin, rot)


def _proj_body(c_ref, wo_ref, out_ref):
    out_ref[:] = jnp.dot(c_ref[:], wo_ref[:],
                         preferred_element_type=jnp.float32).astype(jnp.bfloat16)


def _project(ctx, wo):
    return pl.pallas_call(
        _proj_body,
        grid=(N_DEV,),
        in_specs=[
            pl.BlockSpec((SQ, D), lambda b: (b, 0)),
            pl.BlockSpec((D, D), lambda b: (0, 0)),
        ],
        out_specs=pl.BlockSpec((SQ, D), lambda b: (b, 0)),
        out_shape=jax.ShapeDtypeStruct((N_DEV * SQ, D), jnp.bfloat16),
        compiler_params=pltpu.CompilerParams(
            vmem_limit_bytes=60 * 1024 * 1024,
        ),
    )(ctx, wo)


def _rs_body(p_ref, out_ref, sbuf_ref, comm_ref, send_sems, recv_sems):
    my = lax.axis_index("i")
    left = lax.rem(my + 3, N_DEV)
    right = lax.rem(my + 1, N_DEV)

    barrier = pltpu.get_barrier_semaphore()
    for nbr in (left, right):
        pl.semaphore_signal(
            barrier, inc=1, device_id=(nbr,),
            device_id_type=pl.DeviceIdType.MESH,
        )
    pl.semaphore_wait(barrier, 2)

    for s in range(N_DEV - 1):
        if s == 0:
            src = p_ref.at[pl.ds(1 * SQ, SQ), :]
        else:
            src = sbuf_ref.at[s - 1]
        rdma = pltpu.make_async_remote_copy(
            src_ref=src,
            dst_ref=comm_ref.at[s],
            send_sem=send_sems.at[s],
            recv_sem=recv_sems.at[s],
            device_id=(right,),
            device_id_type=pl.DeviceIdType.MESH,
        )
        rdma.start()
        rdma.wait()
        o_recv = (s + 2) % N_DEV
        mine = p_ref[pl.ds(o_recv * SQ, SQ), :].astype(jnp.float32)
        acc = comm_ref[s, :, :].astype(jnp.float32) + mine
        if s < N_DEV - 2:
            sbuf_ref[s, :, :] = acc.astype(jnp.bfloat16)
        else:
            out_ref[:] = acc


def _reduce_scatter(partial):
    return pl.pallas_call(
        _rs_body,
        out_shape=jax.ShapeDtypeStruct((SQ, D), jnp.float32),
        in_specs=[pl.BlockSpec(memory_space=pltpu.VMEM)],
        out_specs=pl.BlockSpec(memory_space=pltpu.VMEM),
        scratch_shapes=[
            pltpu.VMEM((N_DEV - 2, SQ, D), jnp.bfloat16),
            pltpu.VMEM((N_DEV - 1, SQ, D), jnp.bfloat16),
            pltpu.SemaphoreType.DMA((N_DEV - 1,)),
            pltpu.SemaphoreType.DMA((N_DEV - 1,)),
        ],
        compiler_params=pltpu.CompilerParams(
            collective_id=1, vmem_limit_bytes=60 * 1024 * 1024,
        ),
    )(partial)


def kernel(x, Wq, Wk, Wv, Wo):
    x2 = x.reshape(SQ, D).astype(jnp.bfloat16)
    ctx = _attention(
        x2,
        Wq.astype(jnp.bfloat16),
        Wk.astype(jnp.bfloat16),
        Wv.astype(jnp.bfloat16),
    )
    partial = _project(ctx, Wo.astype(jnp.bfloat16))
    out = _reduce_scatter(partial)
    return out.reshape(1, SQ, D)


# device time: 547543 ns/iter; 1.5590x vs baseline; 1.1006x over previous
import functools

import numpy as np

import jax
import jax.numpy as jnp
from jax import lax
from jax.experimental import pallas as pl
from jax.experimental.pallas import tpu as pltpu

N_DEV = 4
SQ = 2048
D = 1024
DH = 128
H_LOC = 8
SCALE = 0.08838834764831843

_inv = 1.0 / (10000.0 ** (np.arange(0, DH, 2) / DH))
_pos = np.arange(SQ)[:, None] * _inv[None, :]
_COS = np.repeat(np.cos(_pos), 2, axis=-1).astype(np.float32)
_SIN = np.repeat(np.sin(_pos), 2, axis=-1).astype(np.float32)
_ROT = np.zeros((DH, DH), dtype=np.float32)
for _k in range(DH // 2):
    _ROT[2 * _k + 1, 2 * _k] = -1.0
    _ROT[2 * _k, 2 * _k + 1] = 1.0


def _attn_body(x_ref, wq_ref, wk_ref, wv_ref, cos_ref, sin_ref, rot_ref,
               out_ref, comm_ref, send_sems, recv_sems):
    b = pl.program_id(0)
    h = pl.program_id(1)
    my = lax.axis_index("i")
    left = lax.rem(my + 3, N_DEV)
    right = lax.rem(my + 1, N_DEV)

    @pl.when(jnp.logical_and(b == 0, h == 0))
    def _():
        barrier = pltpu.get_barrier_semaphore()
        for nbr in (left, right):
            pl.semaphore_signal(
                barrier, inc=1, device_id=(nbr,),
                device_id_type=pl.DeviceIdType.MESH,
            )
        pl.semaphore_wait(barrier, 2)
        hop0 = pltpu.make_async_remote_copy(
            src_ref=x_ref,
            dst_ref=comm_ref.at[0],
            send_sem=send_sems.at[0],
            recv_sem=recv_sems.at[0],
            device_id=(right,),
            device_id_type=pl.DeviceIdType.MESH,
        )
        hop0.start()

    for bb in range(1, N_DEV):
        @pl.when(jnp.logical_and(b == bb, h == 0))
        def _(bb=bb):
            prev = pltpu.make_async_remote_copy(
                src_ref=comm_ref.at[bb - 1],
                dst_ref=comm_ref.at[bb - 1],
                send_sem=send_sems.at[bb - 1],
                recv_sem=recv_sems.at[bb - 1],
                device_id=(right,),
                device_id_type=pl.DeviceIdType.MESH,
            )
            prev.wait_send()
            prev.wait_recv()
            if bb <= N_DEV - 2:
                nxt = pltpu.make_async_remote_copy(
                    src_ref=comm_ref.at[bb - 1],
                    dst_ref=comm_ref.at[bb],
                    send_sem=send_sems.at[bb],
                    recv_sem=recv_sems.at[bb],
                    device_id=(right,),
                    device_id_type=pl.DeviceIdType.MESH,
                )
                nxt.start()

    xb = jnp.where(b == 0, x_ref[:], comm_ref[jnp.maximum(b - 1, 0)])
    cos = cos_ref[:]
    sin = sin_ref[:]
    rot = rot_ref[:]

    q = jnp.dot(xb, wq_ref[:], preferred_element_type=jnp.float32)
    k = jnp.dot(xb, wk_ref[:], preferred_element_type=jnp.float32)
    v = jnp.dot(xb, wv_ref[:], preferred_element_type=jnp.float32)
    v = v.astype(jnp.bfloat16)

    qrot = jnp.dot(q.astype(jnp.bfloat16), rot,
                   preferred_element_type=jnp.float32)
    krot = jnp.dot(k.astype(jnp.bfloat16), rot,
                   preferred_element_type=jnp.float32)
    qr = (q * cos + qrot * sin).astype(jnp.bfloat16)
    kr = (k * cos + krot * sin).astype(jnp.bfloat16)

    s = lax.dot_general(qr, kr, (((1,), (1,)), ((), ())),
                        preferred_element_type=jnp.float32)
    w = jnp.exp(s * SCALE)
    denom = jnp.sum(w, axis=-1, keepdims=True)
    ctx = jnp.dot(w.astype(jnp.bfloat16), v,
                  preferred_element_type=jnp.float32)
    out_ref[:] = (ctx * (1.0 / denom)).astype(jnp.bfloat16)


def _attention(x2, wq, wk, wv):
    cos = jnp.asarray(_COS)
    sin = jnp.asarray(_SIN)
    rot = jnp.asarray(_ROT, dtype=jnp.bfloat16)
    return pl.pallas_call(
        _attn_body,
        grid=(N_DEV, H_LOC),
        in_specs=[
            pl.BlockSpec((SQ, D), lambda b, h: (0, 0)),
            pl.BlockSpec((D, DH), lambda b, h: (0, h)),
            pl.BlockSpec((D, DH), lambda b, h: (0, h)),
            pl.BlockSpec((D, DH), lambda b, h: (0, h)),
            pl.BlockSpec((SQ, DH), lambda b, h: (0, 0)),
            pl.BlockSpec((SQ, DH), lambda b, h: (0, 0)),
            pl.BlockSpec((DH, DH), lambda b, h: (0, 0)),
        ],
        out_specs=pl.BlockSpec((SQ, DH), lambda b, h: (b, h)),
        out_shape=jax.ShapeDtypeStruct((N_DEV * SQ, D), jnp.bfloat16),
        scratch_shapes=[
            pltpu.VMEM((N_DEV - 1, SQ, D), jnp.bfloat16),
            pltpu.SemaphoreType.DMA((N_DEV - 1,)),
            pltpu.SemaphoreType.DMA((N_DEV - 1,)),
        ],
        compiler_params=pltpu.CompilerParams(
            collective_id=0, vmem_limit_bytes=60 * 1024 * 1024,
        ),
    )(x2, wq, wk, wv, cos, sin, rot)


def _rs_body(p_ref, wo_ref, out_ref, sbuf_ref, comm_ref, send_sems, recv_sems):
    my = lax.axis_index("i")
    left = lax.rem(my + 3, N_DEV)
    diag = lax.rem(my + 2, N_DEV)
    right = lax.rem(my + 1, N_DEV)
    targets = (left, diag, right)

    barrier = pltpu.get_barrier_semaphore()
    for nbr in targets:
        pl.semaphore_signal(
            barrier, inc=1, device_id=(nbr,),
            device_id_type=pl.DeviceIdType.MESH,
        )
    pl.semaphore_wait(barrier, 3)

    wo = wo_ref[:]
    rdmas = []
    for o in (1, 2, 3):
        proj = jnp.dot(p_ref[pl.ds(o * SQ, SQ), :], wo,
                       preferred_element_type=jnp.float32)
        sbuf_ref[o - 1, :, :] = proj.astype(jnp.bfloat16)
        rdma = pltpu.make_async_remote_copy(
            src_ref=sbuf_ref.at[o - 1],
            dst_ref=comm_ref.at[o - 1],
            send_sem=send_sems.at[o - 1],
            recv_sem=recv_sems.at[o - 1],
            device_id=(targets[o - 1],),
            device_id_type=pl.DeviceIdType.MESH,
        )
        rdma.start()
        rdmas.append(rdma)

    acc = jnp.dot(p_ref[pl.ds(0, SQ), :], wo,
                  preferred_element_type=jnp.float32)
    for rdma in rdmas:
        rdma.wait_send()
        rdma.wait_recv()
    for k in range(N_DEV - 1):
        acc = acc + comm_ref[k, :, :].astype(jnp.float32)
    out_ref[:] = acc


def _rs_proj(ctx, wo):
    return pl.pallas_call(
        _rs_body,
        out_shape=jax.ShapeDtypeStruct((SQ, D), jnp.float32),
        in_specs=[
            pl.BlockSpec(memory_space=pltpu.VMEM),
            pl.BlockSpec(memory_space=pltpu.VMEM),
        ],
        out_specs=pl.BlockSpec(memory_space=pltpu.VMEM),
        scratch_shapes=[
            pltpu.VMEM((N_DEV - 1, SQ, D), jnp.bfloat16),
            pltpu.VMEM((N_DEV - 1, SQ, D), jnp.bfloat16),
            pltpu.SemaphoreType.DMA((N_DEV - 1,)),
            pltpu.SemaphoreType.DMA((N_DEV - 1,)),
        ],
        compiler_params=pltpu.CompilerParams(
            collective_id=1, vmem_limit_bytes=62 * 1024 * 1024,
        ),
    )(ctx, wo)


def kernel(x, Wq, Wk, Wv, Wo):
    x2 = x.reshape(SQ, D).astype(jnp.bfloat16)
    ctx = _attention(
        x2,
        Wq.astype(jnp.bfloat16),
        Wk.astype(jnp.bfloat16),
        Wv.astype(jnp.bfloat16),
    )
    out = _rs_proj(ctx, Wo.astype(jnp.bfloat16))
    return out.reshape(1, SQ, D)


# device time: 530583 ns/iter; 1.6088x vs baseline; 1.0320x over previous
import functools

import numpy as np

import jax
import jax.numpy as jnp
from jax import lax
from jax.experimental import pallas as pl
from jax.experimental.pallas import tpu as pltpu

N_DEV = 4
SQ = 2048
D = 1024
DH = 128
H_LOC = 8
SCALE = 0.08838834764831843
LOG2E = 1.4426950408889634

_inv = 1.0 / (10000.0 ** (np.arange(0, DH, 2) / DH))
_pos = np.arange(SQ)[:, None] * _inv[None, :]
_COS = np.repeat(np.cos(_pos), 2, axis=-1).astype(np.float32)
_SIN = np.repeat(np.sin(_pos), 2, axis=-1).astype(np.float32)
_ROT = np.zeros((DH, DH), dtype=np.float32)
for _k in range(DH // 2):
    _ROT[2 * _k + 1, 2 * _k] = -1.0
    _ROT[2 * _k, 2 * _k + 1] = 1.0


def _attn_body(x_ref, wq_ref, wk_ref, wv_ref, cosq_ref, sinq_ref, cos_ref,
               sin_ref, rot_ref, out_ref, comm_ref, send_sems, recv_sems):
    b = pl.program_id(0)
    h = pl.program_id(1)
    my = lax.axis_index("i")
    left = lax.rem(my + 3, N_DEV)
    right = lax.rem(my + 1, N_DEV)

    @pl.when(jnp.logical_and(b == 0, h == 0))
    def _():
        barrier = pltpu.get_barrier_semaphore()
        for nbr in (left, right):
            pl.semaphore_signal(
                barrier, inc=1, device_id=(nbr,),
                device_id_type=pl.DeviceIdType.MESH,
            )
        pl.semaphore_wait(barrier, 2)
        comm_ref[N_DEV - 1, :, :] = x_ref[:]
        hop0 = pltpu.make_async_remote_copy(
            src_ref=x_ref,
            dst_ref=comm_ref.at[0],
            send_sem=send_sems.at[0],
            recv_sem=recv_sems.at[0],
            device_id=(right,),
            device_id_type=pl.DeviceIdType.MESH,
        )
        hop0.start()

    for bb in range(1, N_DEV):
        @pl.when(jnp.logical_and(b == bb, h == 0))
        def _(bb=bb):
            prev = pltpu.make_async_remote_copy(
                src_ref=comm_ref.at[bb - 1],
                dst_ref=comm_ref.at[bb - 1],
                send_sem=send_sems.at[bb - 1],
                recv_sem=recv_sems.at[bb - 1],
                device_id=(right,),
                device_id_type=pl.DeviceIdType.MESH,
            )
            prev.wait_send()
            prev.wait_recv()
            if bb <= N_DEV - 2:
                nxt = pltpu.make_async_remote_copy(
                    src_ref=comm_ref.at[bb - 1],
                    dst_ref=comm_ref.at[bb],
                    send_sem=send_sems.at[bb],
                    recv_sem=recv_sems.at[bb],
                    device_id=(right,),
                    device_id_type=pl.DeviceIdType.MESH,
                )
                nxt.start()

    xb = comm_ref[lax.rem(b + N_DEV - 1, N_DEV)]
    rot = rot_ref[:]

    q = jnp.dot(xb, wq_ref[:], preferred_element_type=jnp.float32)
    k = jnp.dot(xb, wk_ref[:], preferred_element_type=jnp.float32)
    v = jnp.dot(xb, wv_ref[:], preferred_element_type=jnp.float32)
    v = v.astype(jnp.bfloat16)

    qrot = jnp.dot(q.astype(jnp.bfloat16), rot,
                   preferred_element_type=jnp.float32)
    krot = jnp.dot(k.astype(jnp.bfloat16), rot,
                   preferred_element_type=jnp.float32)
    qr = (q * cosq_ref[:] + qrot * sinq_ref[:]).astype(jnp.bfloat16)
    kr = (k * cos_ref[:] + krot * sin_ref[:]).astype(jnp.bfloat16)

    s = lax.dot_general(qr, kr, (((1,), (1,)), ((), ())),
                        preferred_element_type=jnp.float32)
    w = jnp.exp2(s)
    denom = jnp.sum(w, axis=-1, keepdims=True)
    ctx = jnp.dot(w.astype(jnp.bfloat16), v,
                  preferred_element_type=jnp.float32)
    out_ref[:] = (ctx * (1.0 / denom)).astype(jnp.bfloat16)


def _attention(x2, wq, wk, wv):
    cosq = jnp.asarray(_COS * (SCALE * LOG2E))
    sinq = jnp.asarray(_SIN * (SCALE * LOG2E))
    cos = jnp.asarray(_COS)
    sin = jnp.asarray(_SIN)
    rot = jnp.asarray(_ROT, dtype=jnp.bfloat16)
    return pl.pallas_call(
        _attn_body,
        grid=(N_DEV, H_LOC),
        in_specs=[
            pl.BlockSpec((SQ, D), lambda b, h: (0, 0)),
            pl.BlockSpec((D, DH), lambda b, h: (0, h)),
            pl.BlockSpec((D, DH), lambda b, h: (0, h)),
            pl.BlockSpec((D, DH), lambda b, h: (0, h)),
            pl.BlockSpec((SQ, DH), lambda b, h: (0, 0)),
            pl.BlockSpec((SQ, DH), lambda b, h: (0, 0)),
            pl.BlockSpec((SQ, DH), lambda b, h: (0, 0)),
            pl.BlockSpec((SQ, DH), lambda b, h: (0, 0)),
            pl.BlockSpec((DH, DH), lambda b, h: (0, 0)),
        ],
        out_specs=pl.BlockSpec((SQ, DH), lambda b, h: (b, h)),
        out_shape=jax.ShapeDtypeStruct((N_DEV * SQ, D), jnp.bfloat16),
        scratch_shapes=[
            pltpu.VMEM((N_DEV, SQ, D), jnp.bfloat16),
            pltpu.SemaphoreType.DMA((N_DEV - 1,)),
            pltpu.SemaphoreType.DMA((N_DEV - 1,)),
        ],
        compiler_params=pltpu.CompilerParams(
            collective_id=0, vmem_limit_bytes=60 * 1024 * 1024,
        ),
    )(x2, wq, wk, wv, cosq, sinq, cos, sin, rot)


def _rs_body(p_ref, wo_ref, out_ref, sbuf_ref, comm_ref, send_sems, recv_sems):
    my = lax.axis_index("i")
    left = lax.rem(my + 3, N_DEV)
    diag = lax.rem(my + 2, N_DEV)
    right = lax.rem(my + 1, N_DEV)
    targets = (left, diag, right)

    barrier = pltpu.get_barrier_semaphore()
    for nbr in targets:
        pl.semaphore_signal(
            barrier, inc=1, device_id=(nbr,),
            device_id_type=pl.DeviceIdType.MESH,
        )
    pl.semaphore_wait(barrier, 3)

    wo = wo_ref[:]
    rdmas = []
    for o in (1, 2, 3):
        proj = jnp.dot(p_ref[pl.ds(o * SQ, SQ), :], wo,
                       preferred_element_type=jnp.float32)
        sbuf_ref[o - 1, :, :] = proj.astype(jnp.bfloat16)
        rdma = pltpu.make_async_remote_copy(
            src_ref=sbuf_ref.at[o - 1],
            dst_ref=comm_ref.at[o - 1],
            send_sem=send_sems.at[o - 1],
            recv_sem=recv_sems.at[o - 1],
            device_id=(targets[o - 1],),
            device_id_type=pl.DeviceIdType.MESH,
        )
        rdma.start()
        rdmas.append(rdma)

    acc = jnp.dot(p_ref[pl.ds(0, SQ), :], wo,
                  preferred_element_type=jnp.float32)
    for rdma in rdmas:
        rdma.wait_send()
        rdma.wait_recv()
    for k in range(N_DEV - 1):
        acc = acc + comm_ref[k, :, :].astype(jnp.float32)
    out_ref[:] = acc


def _rs_proj(ctx, wo):
    return pl.pallas_call(
        _rs_body,
        out_shape=jax.ShapeDtypeStruct((SQ, D), jnp.float32),
        in_specs=[
            pl.BlockSpec(memory_space=pltpu.VMEM),
            pl.BlockSpec(memory_space=pltpu.VMEM),
        ],
        out_specs=pl.BlockSpec(memory_space=pltpu.VMEM),
        scratch_shapes=[
            pltpu.VMEM((N_DEV - 1, SQ, D), jnp.bfloat16),
            pltpu.VMEM((N_DEV - 1, SQ, D), jnp.bfloat16),
            pltpu.SemaphoreType.DMA((N_DEV - 1,)),
            pltpu.SemaphoreType.DMA((N_DEV - 1,)),
        ],
        compiler_params=pltpu.CompilerParams(
            collective_id=1, vmem_limit_bytes=62 * 1024 * 1024,
        ),
    )(ctx, wo)


def kernel(x, Wq, Wk, Wv, Wo):
    x2 = x.reshape(SQ, D).astype(jnp.bfloat16)
    ctx = _attention(
        x2,
        Wq.astype(jnp.bfloat16),
        Wk.astype(jnp.bfloat16),
        Wv.astype(jnp.bfloat16),
    )
    out = _rs_proj(ctx, Wo.astype(jnp.bfloat16))
    return out.reshape(1, SQ, D)
